# baseline (speedup 1.0000x reference)
import functools

import jax
import jax.numpy as jnp
import numpy as np
from jax import lax
from jax.experimental import pallas as pl
from jax.experimental.pallas import tpu as pltpu
from jax.experimental.pallas import tpu_sc as plsc

V = 100000
D = 128
B = 500000

NC, NS = 2, 16
NW = NC * NS
L = 16

C = 200
SUB = 100
KSUB = C // SUB
NGBUF = 4
NCHUNK = B // C
CPW = (NCHUNK + NW - 1) // NW

FR = 160
FCH = V // FR

_MESH = plsc.VectorSubcoreMesh(
    core_axis_name="c", subcore_axis_name="s", num_cores=NC, num_subcores=NS
)


@functools.partial(
    pl.kernel,
    out_type=jax.ShapeDtypeStruct((V, D), jnp.float32),
    mesh=_MESH,
    scratch_types=[
        *[pltpu.VMEM((FR, D), jnp.float32) for _ in range(6)],
        *[pltpu.SemaphoreType.DMA for _ in range(4)],
    ],
)
def _fuse_k(nf_hbm, mem_hbm, fused_hbm, nf0, mem0, fs0, nf1, mem1, fs1,
            sg0, sg1, ss0, ss1):
    wid = lax.axis_index("s") * NC + lax.axis_index("c")
    n_mine = (FCH - wid + NW - 1) // NW
    bufs = ((nf0, mem0, fs0, sg0, ss0), (nf1, mem1, fs1, sg1, ss1))

    def fire_loads(i, nfb, memb, semg, make_only):
        mk = pltpu.make_async_copy if make_only else pltpu.async_copy
        r = (wid + i * NW) * FR
        return [
            mk(nf_hbm.at[pl.ds(r, FR), :], nfb, semg),
            mk(mem_hbm.at[pl.ds(r, FR), :], memb, semg),
        ]

    for b in (0, 1):
        nfb, memb, _, semg, _ = bufs[b]
        fire_loads(b, nfb, memb, semg, make_only=False)

    def body(g, carry):
        for b in (0, 1):
            i = 2 * g + b
            nfb, memb, fsb, semg, sems = bufs[b]

            @pl.when(i < n_mine)
            def _process():
                for cp in fire_loads(i, nfb, memb, semg, make_only=True):
                    cp.wait()

                @pl.when(i >= 2)
                def _drain_store():
                    pltpu.make_async_copy(
                        fsb, fused_hbm.at[pl.ds(0, FR), :], sems
                    ).wait()

                def row(r, carry2):
                    for h in range(D // L):
                        fsb[r, pl.ds(L * h, L)] = (
                            nfb[r, pl.ds(L * h, L)] + memb[r, pl.ds(L * h, L)]
                        )
                    return carry2

                lax.fori_loop(0, FR, row, 0)
                pltpu.async_copy(
                    fsb, fused_hbm.at[pl.ds((wid + i * NW) * FR, FR), :], sems
                )

                @pl.when(i + 2 < n_mine)
                def _refill():
                    fire_loads(i + 2, nfb, memb, semg, make_only=False)

        return carry

    lax.fori_loop(0, (n_mine + 1) // 2, body, 0)
    for b in (0, 1):
        pltpu.make_async_copy(bufs[b][2], fused_hbm.at[pl.ds(0, FR), :],
                              bufs[b][4]).wait()


_PERM = np.zeros((NW, CPW), dtype=np.int32)
for _w in range(NW):
    _js = np.arange(_w, NCHUNK, NW, dtype=np.int32)
    _PERM[_w, : len(_js)] = _js
_PERM_J = _PERM.reshape(-1)


@functools.partial(
    pl.kernel,
    out_type=jax.ShapeDtypeStruct((B, D), jnp.float32),
    mesh=_MESH,
    scratch_types=[
        pltpu.VMEM((CPW, KSUB, SUB), jnp.int32),
        *[pltpu.VMEM((C, D), jnp.float32) for _ in range(NGBUF)],
        *[pltpu.SemaphoreType.DMA for _ in range(2 * NGBUF)],
    ],
)
def _gather_k(table_hbm, idx_hbm, out_hbm, idx_all, *refs):
    gbuf = refs[:NGBUF]
    sg = refs[NGBUF : 2 * NGBUF]
    ss = refs[2 * NGBUF :]
    wid = lax.axis_index("s") * NC + lax.axis_index("c")
    n_mine = (NCHUNK - wid + NW - 1) // NW

    pltpu.sync_copy(idx_hbm.at[wid], idx_all)

    def gather_copies(i, b, make_only):
        mk = pltpu.make_async_copy if make_only else pltpu.async_copy
        return [
            mk(table_hbm.at[idx_all.at[i, k]],
               gbuf[b].at[pl.ds(k * SUB, SUB), :], sg[b])
            for k in range(KSUB)
        ]

    for b in range(NGBUF):
        gather_copies(b, b, make_only=False)

    def body(g, carry):
        for b in range(NGBUF):
            t = NGBUF * g + b

            @pl.when(t < n_mine)
            def _process():
                for cp in gather_copies(t, b, make_only=True):
                    cp.wait()
                j = wid + t * NW
                st = pltpu.async_copy(gbuf[b], out_hbm.at[pl.ds(j * C, C), :],
                                      ss[b])
                st.wait()

                @pl.when(t + NGBUF < n_mine)
                def _refill():
                    gather_copies(t + NGBUF, b, make_only=False)

        return carry

    lax.fori_loop(0, (n_mine + NGBUF - 1) // NGBUF, body, 0)


def kernel(node_features, memory, source_nodes, timestamps, time_w, time_b):
    del timestamps, time_w, time_b
    fused = _fuse_k(node_features, memory)
    idx = source_nodes.astype(jnp.int32).reshape(NCHUNK, C)
    idx = jnp.take(idx, _PERM_J, axis=0).reshape(NW, CPW, KSUB, SUB)
    return _gather_k(fused, idx)


# speedup vs baseline: 1.0294x; 1.0294x over previous
import functools

import jax
import jax.numpy as jnp
import numpy as np
from jax import lax
from jax.experimental import pallas as pl
from jax.experimental.pallas import tpu as pltpu
from jax.experimental.pallas import tpu_sc as plsc

V = 100000
D = 128
B = 500000

NC, NS = 2, 16
NW = NC * NS
L = 16

C = 160
SUB = 80
KSUB = C // SUB
NGBUF = 6
NCHUNK = B // C

FR = 160
FCH = V // FR

_MESH = plsc.VectorSubcoreMesh(
    core_axis_name="c", subcore_axis_name="s", num_cores=NC, num_subcores=NS
)


@functools.partial(
    pl.kernel,
    out_type=jax.ShapeDtypeStruct((V, D), jnp.float32),
    mesh=_MESH,
    scratch_types=[
        *[pltpu.VMEM((FR, D), jnp.float32) for _ in range(6)],
        *[pltpu.SemaphoreType.DMA for _ in range(4)],
    ],
)
def _fuse_k(nf_hbm, mem_hbm, fused_hbm, nf0, mem0, fs0, nf1, mem1, fs1,
            sg0, sg1, ss0, ss1):
    wid = lax.axis_index("s") * NC + lax.axis_index("c")
    n_mine = (FCH - wid + NW - 1) // NW
    bufs = ((nf0, mem0, fs0, sg0, ss0), (nf1, mem1, fs1, sg1, ss1))

    def fire_loads(i, nfb, memb, semg, make_only):
        mk = pltpu.make_async_copy if make_only else pltpu.async_copy
        r = (wid + i * NW) * FR
        return [
            mk(nf_hbm.at[pl.ds(r, FR), :], nfb, semg),
            mk(mem_hbm.at[pl.ds(r, FR), :], memb, semg),
        ]

    for b in (0, 1):
        nfb, memb, _, semg, _ = bufs[b]
        fire_loads(b, nfb, memb, semg, make_only=False)

    def body(g, carry):
        for b in (0, 1):
            i = 2 * g + b
            nfb, memb, fsb, semg, sems = bufs[b]

            @pl.when(i < n_mine)
            def _process():
                for cp in fire_loads(i, nfb, memb, semg, make_only=True):
                    cp.wait()

                @pl.when(i >= 2)
                def _drain_store():
                    pltpu.make_async_copy(
                        fsb, fused_hbm.at[pl.ds(0, FR), :], sems
                    ).wait()

                def row(r, carry2):
                    for h in range(D // L):
                        fsb[r, pl.ds(L * h, L)] = (
                            nfb[r, pl.ds(L * h, L)] + memb[r, pl.ds(L * h, L)]
                        )
                    return carry2

                lax.fori_loop(0, FR, row, 0)
                pltpu.async_copy(
                    fsb, fused_hbm.at[pl.ds((wid + i * NW) * FR, FR), :], sems
                )

                @pl.when(i + 2 < n_mine)
                def _refill():
                    fire_loads(i + 2, nfb, memb, semg, make_only=False)

        return carry

    lax.fori_loop(0, (n_mine + 1) // 2, body, 0)
    for b in (0, 1):
        pltpu.make_async_copy(bufs[b][2], fused_hbm.at[pl.ds(0, FR), :],
                              bufs[b][4]).wait()


@functools.partial(
    pl.kernel,
    out_type=jax.ShapeDtypeStruct((B, D), jnp.float32),
    mesh=_MESH,
    scratch_types=[
        *[pltpu.VMEM((KSUB, SUB), jnp.int32) for _ in range(NGBUF)],
        *[pltpu.VMEM((C, D), jnp.float32) for _ in range(NGBUF)],
        *[pltpu.SemaphoreType.DMA for _ in range(2 * NGBUF)],
    ],
)
def _gather_k(table_hbm, idx_hbm, out_hbm, *refs):
    ibuf = refs[:NGBUF]
    gbuf = refs[NGBUF : 2 * NGBUF]
    sg = refs[2 * NGBUF : 3 * NGBUF]
    ss = refs[3 * NGBUF :]
    wid = lax.axis_index("s") * NC + lax.axis_index("c")
    n_mine = (NCHUNK - wid + NW - 1) // NW

    def gather_copies(t, b, make_only):
        mk = pltpu.make_async_copy if make_only else pltpu.async_copy
        if not make_only:
            j = wid + t * NW
            pltpu.sync_copy(idx_hbm.at[pl.ds(j * KSUB, KSUB), :], ibuf[b])
        return [
            mk(table_hbm.at[ibuf[b].at[k]],
               gbuf[b].at[pl.ds(k * SUB, SUB), :], sg[b])
            for k in range(KSUB)
        ]

    for b in range(NGBUF):
        gather_copies(b, b, make_only=False)

    def body(g, carry):
        for b in range(NGBUF):
            t = NGBUF * g + b

            @pl.when(t < n_mine)
            def _process():
                for cp in gather_copies(t, b, make_only=True):
                    cp.wait()
                j = wid + t * NW
                st = pltpu.async_copy(gbuf[b], out_hbm.at[pl.ds(j * C, C), :],
                                      ss[b])
                st.wait()

                @pl.when(t + NGBUF < n_mine)
                def _refill():
                    gather_copies(t + NGBUF, b, make_only=False)

        return carry

    lax.fori_loop(0, (n_mine + NGBUF - 1) // NGBUF, body, 0)


def kernel(node_features, memory, source_nodes, timestamps, time_w, time_b):
    del timestamps, time_w, time_b
    fused = _fuse_k(node_features, memory)
    idx = source_nodes.astype(jnp.int32).reshape(NCHUNK * KSUB, SUB)
    return _gather_k(fused, idx)


# speedup vs baseline: 1.0489x; 1.0189x over previous
import functools

import jax
import jax.numpy as jnp
import numpy as np
from jax import lax
from jax.experimental import pallas as pl
from jax.experimental.pallas import tpu as pltpu
from jax.experimental.pallas import tpu_sc as plsc

V = 100000
D = 128
B = 500000

NC, NS = 2, 16
NW = NC * NS
L = 16

C = 200
SUB = 100
KSUB = C // SUB
NGBUF = 4
NCHUNK = B // C
CPW = (NCHUNK + NW - 1) // NW

FR = 160
FCH = V // FR

_MESH = plsc.VectorSubcoreMesh(
    core_axis_name="c", subcore_axis_name="s", num_cores=NC, num_subcores=NS
)


@functools.partial(
    pl.kernel,
    out_type=jax.ShapeDtypeStruct((V, D), jnp.float32),
    mesh=_MESH,
    scratch_types=[
        *[pltpu.VMEM((FR, D), jnp.float32) for _ in range(6)],
        *[pltpu.SemaphoreType.DMA for _ in range(4)],
    ],
)
def _fuse_k(nf_hbm, mem_hbm, fused_hbm, nf0, mem0, fs0, nf1, mem1, fs1,
            sg0, sg1, ss0, ss1):
    wid = lax.axis_index("s") * NC + lax.axis_index("c")
    n_mine = (FCH - wid + NW - 1) // NW
    bufs = ((nf0, mem0, fs0, sg0, ss0), (nf1, mem1, fs1, sg1, ss1))

    def fire_loads(i, nfb, memb, semg, make_only):
        mk = pltpu.make_async_copy if make_only else pltpu.async_copy
        r = (wid + i * NW) * FR
        return [
            mk(nf_hbm.at[pl.ds(r, FR), :], nfb, semg),
            mk(mem_hbm.at[pl.ds(r, FR), :], memb, semg),
        ]

    for b in (0, 1):
        nfb, memb, _, semg, _ = bufs[b]
        fire_loads(b, nfb, memb, semg, make_only=False)

    def body(g, carry):
        for b in (0, 1):
            i = 2 * g + b
            nfb, memb, fsb, semg, sems = bufs[b]

            @pl.when(i < n_mine)
            def _process():
                for cp in fire_loads(i, nfb, memb, semg, make_only=True):
                    cp.wait()

                @pl.when(i >= 2)
                def _drain_store():
                    pltpu.make_async_copy(
                        fsb, fused_hbm.at[pl.ds(0, FR), :], sems
                    ).wait()

                def row(r, carry2):
                    for h in range(D // L):
                        fsb[r, pl.ds(L * h, L)] = (
                            nfb[r, pl.ds(L * h, L)] + memb[r, pl.ds(L * h, L)]
                        )
                    return carry2

                lax.fori_loop(0, FR, row, 0)
                pltpu.async_copy(
                    fsb, fused_hbm.at[pl.ds((wid + i * NW) * FR, FR), :], sems
                )

                @pl.when(i + 2 < n_mine)
                def _refill():
                    fire_loads(i + 2, nfb, memb, semg, make_only=False)

        return carry

    lax.fori_loop(0, (n_mine + 1) // 2, body, 0)
    for b in (0, 1):
        pltpu.make_async_copy(bufs[b][2], fused_hbm.at[pl.ds(0, FR), :],
                              bufs[b][4]).wait()


@functools.partial(
    pl.kernel,
    out_type=jax.ShapeDtypeStruct((B, D), jnp.float32),
    mesh=_MESH,
    scratch_types=[
        pltpu.VMEM((CPW, KSUB, SUB), jnp.int32),
        *[pltpu.VMEM((C, D), jnp.float32) for _ in range(NGBUF)],
        *[pltpu.SemaphoreType.DMA for _ in range(2 * NGBUF)],
        pltpu.SemaphoreType.DMA,
    ],
)
def _gather_k(table_hbm, idx_hbm, out_hbm, idx_all, *refs):
    gbuf = refs[:NGBUF]
    sg = refs[NGBUF : 2 * NGBUF]
    ss = refs[2 * NGBUF : 3 * NGBUF]
    si = refs[3 * NGBUF]
    wid = lax.axis_index("s") * NC + lax.axis_index("c")
    n_mine = (NCHUNK - wid + NW - 1) // NW

    pf = []
    for i in range(CPW):
        jj = wid + i * NW
        j = lax.select(jj < NCHUNK, jj, wid)
        pf.append(pltpu.async_copy(
            idx_hbm.at[pl.ds(j * KSUB, KSUB), :], idx_all.at[i], si))
    for cp in pf:
        cp.wait()

    def gather_copies(t, b, make_only):
        mk = pltpu.make_async_copy if make_only else pltpu.async_copy
        return [
            mk(table_hbm.at[idx_all.at[t, k]],
               gbuf[b].at[pl.ds(k * SUB, SUB), :], sg[b])
            for k in range(KSUB)
        ]

    for b in range(NGBUF):
        gather_copies(b, b, make_only=False)

    def body(g, carry):
        for b in range(NGBUF):
            t = NGBUF * g + b

            @pl.when(t < n_mine)
            def _process():
                for cp in gather_copies(t, b, make_only=True):
                    cp.wait()
                j = wid + t * NW
                st = pltpu.async_copy(gbuf[b], out_hbm.at[pl.ds(j * C, C), :],
                                      ss[b])
                st.wait()

                @pl.when(t + NGBUF < n_mine)
                def _refill():
                    gather_copies(t + NGBUF, b, make_only=False)

        return carry

    lax.fori_loop(0, (n_mine + NGBUF - 1) // NGBUF, body, 0)


def kernel(node_features, memory, source_nodes, timestamps, time_w, time_b):
    del timestamps, time_w, time_b
    fused = _fuse_k(node_features, memory)
    idx = source_nodes.astype(jnp.int32).reshape(NCHUNK * KSUB, SUB)
    return _gather_k(fused, idx)


# speedup vs baseline: 1.0495x; 1.0006x over previous
import functools

import jax
import jax.numpy as jnp
import numpy as np
from jax import lax
from jax.experimental import pallas as pl
from jax.experimental.pallas import tpu as pltpu
from jax.experimental.pallas import tpu_sc as plsc

V = 100000
D = 128
B = 500000

NC, NS = 2, 16
NW = NC * NS
L = 16

C = 160
SUB = 80
KSUB = C // SUB
NGBUF = 5
NCHUNK = B // C
CPW = (NCHUNK + NW - 1) // NW

FR = 160
FCH = V // FR

_MESH = plsc.VectorSubcoreMesh(
    core_axis_name="c", subcore_axis_name="s", num_cores=NC, num_subcores=NS
)


@functools.partial(
    pl.kernel,
    out_type=jax.ShapeDtypeStruct((V, D), jnp.float32),
    mesh=_MESH,
    scratch_types=[
        *[pltpu.VMEM((FR, D), jnp.float32) for _ in range(6)],
        *[pltpu.SemaphoreType.DMA for _ in range(4)],
    ],
)
def _fuse_k(nf_hbm, mem_hbm, fused_hbm, nf0, mem0, fs0, nf1, mem1, fs1,
            sg0, sg1, ss0, ss1):
    wid = lax.axis_index("s") * NC + lax.axis_index("c")
    n_mine = (FCH - wid + NW - 1) // NW
    bufs = ((nf0, mem0, fs0, sg0, ss0), (nf1, mem1, fs1, sg1, ss1))

    def fire_loads(i, nfb, memb, semg, make_only):
        mk = pltpu.make_async_copy if make_only else pltpu.async_copy
        r = (wid + i * NW) * FR
        return [
            mk(nf_hbm.at[pl.ds(r, FR), :], nfb, semg),
            mk(mem_hbm.at[pl.ds(r, FR), :], memb, semg),
        ]

    for b in (0, 1):
        nfb, memb, _, semg, _ = bufs[b]
        fire_loads(b, nfb, memb, semg, make_only=False)

    def body(g, carry):
        for b in (0, 1):
            i = 2 * g + b
            nfb, memb, fsb, semg, sems = bufs[b]

            @pl.when(i < n_mine)
            def _process():
                for cp in fire_loads(i, nfb, memb, semg, make_only=True):
                    cp.wait()

                @pl.when(i >= 2)
                def _drain_store():
                    pltpu.make_async_copy(
                        fsb, fused_hbm.at[pl.ds(0, FR), :], sems
                    ).wait()

                def row(r, carry2):
                    for h in range(D // L):
                        fsb[r, pl.ds(L * h, L)] = (
                            nfb[r, pl.ds(L * h, L)] + memb[r, pl.ds(L * h, L)]
                        )
                    return carry2

                lax.fori_loop(0, FR, row, 0)
                pltpu.async_copy(
                    fsb, fused_hbm.at[pl.ds((wid + i * NW) * FR, FR), :], sems
                )

                @pl.when(i + 2 < n_mine)
                def _refill():
                    fire_loads(i + 2, nfb, memb, semg, make_only=False)

        return carry

    lax.fori_loop(0, (n_mine + 1) // 2, body, 0)
    for b in (0, 1):
        pltpu.make_async_copy(bufs[b][2], fused_hbm.at[pl.ds(0, FR), :],
                              bufs[b][4]).wait()


@functools.partial(
    pl.kernel,
    out_type=jax.ShapeDtypeStruct((B, D), jnp.float32),
    mesh=_MESH,
    scratch_types=[
        pltpu.VMEM((CPW, KSUB, SUB), jnp.int32),
        *[pltpu.VMEM((C, D), jnp.float32) for _ in range(NGBUF)],
        *[pltpu.SemaphoreType.DMA for _ in range(2 * NGBUF)],
        pltpu.SemaphoreType.DMA,
    ],
)
def _gather_k(table_hbm, idx_hbm, out_hbm, idx_all, *refs):
    gbuf = refs[:NGBUF]
    sg = refs[NGBUF : 2 * NGBUF]
    ss = refs[2 * NGBUF : 3 * NGBUF]
    si = refs[3 * NGBUF]
    wid = lax.axis_index("s") * NC + lax.axis_index("c")
    n_mine = (NCHUNK - wid + NW - 1) // NW

    pf = []
    for i in range(CPW):
        jj = wid + i * NW
        j = lax.select(jj < NCHUNK, jj, wid)
        pf.append(pltpu.async_copy(
            idx_hbm.at[pl.ds(j * KSUB, KSUB), :], idx_all.at[i], si))
    for cp in pf:
        cp.wait()

    def gather_copy(t, b, k, make_only):
        mk = pltpu.make_async_copy if make_only else pltpu.async_copy
        return mk(table_hbm.at[idx_all.at[t, k]],
                  gbuf[b].at[pl.ds(k * SUB, SUB), :], sg[b])

    for b in range(NGBUF):
        for k in range(KSUB):
            gather_copy(b, b, k, make_only=False)

    def body(g, carry):
        for b in range(NGBUF):
            t = NGBUF * g + b

            @pl.when(t < n_mine)
            def _process():
                j = wid + t * NW
                sts = []
                for k in range(KSUB):
                    gather_copy(t, b, k, make_only=True).wait()
                    sts.append(pltpu.async_copy(
                        gbuf[b].at[pl.ds(k * SUB, SUB), :],
                        out_hbm.at[pl.ds(j * C + k * SUB, SUB), :], ss[b]))
                for st in sts:
                    st.wait()

                @pl.when(t + NGBUF < n_mine)
                def _refill():
                    for k in range(KSUB):
                        gather_copy(t + NGBUF, b, k, make_only=False)

        return carry

    lax.fori_loop(0, (n_mine + NGBUF - 1) // NGBUF, body, 0)


def kernel(node_features, memory, source_nodes, timestamps, time_w, time_b):
    del timestamps, time_w, time_b
    fused = _fuse_k(node_features, memory)
    idx = source_nodes.astype(jnp.int32).reshape(NCHUNK * KSUB, SUB)
    return _gather_k(fused, idx)
